# baseline (speedup 1.0000x reference)
import functools

import jax
import jax.numpy as jnp
from jax import lax
from jax.experimental import pallas as pl
from jax.experimental.pallas import tpu as pltpu
from jax.experimental.pallas import tpu_sc as plsc

VOCAB = 1_000_000
WORLD = 4
MY_RANK = 1
PART = VOCAB // WORLD
LO = MY_RANK * PART
HI = LO + PART
EMB = 64
BATCH = 4096
SEQ = 200
NTOK = BATCH * SEQ

NC = 2
NS = 16
NW = NC * NS
NB = BATCH // NW
PER_W = NB * SEQ
L = 16
G = PER_W // L
DUMMY = NB


@functools.partial(
    pl.kernel,
    out_type=jax.ShapeDtypeStruct((SEQ, EMB // 8, BATCH // 128, 8, 128),
                                  jnp.float32),
    mesh=plsc.VectorSubcoreMesh(core_axis_name="c", subcore_axis_name="s"),
    compiler_params=pltpu.CompilerParams(
        use_tc_tiling_on_sc=False, needs_layout_passes=False),
    scratch_types=[
        pltpu.VMEM((PER_W,), jnp.int32),
        pltpu.VMEM((PER_W,), jnp.int32),
        pltpu.VMEM((PER_W,), jnp.int32),
        pltpu.VMEM((NB + 1, EMB), jnp.float32),
        pltpu.VMEM((NB + 1, EMB), jnp.float32),
        pltpu.VMEM((EMB // 8, 8, 128), jnp.float32),
        pltpu.VMEM((EMB // 8, 8, 128), jnp.float32),
        pltpu.SemaphoreType.DMA,
        pltpu.SemaphoreType.DMA,
        pltpu.SemaphoreType.DMA,
        pltpu.SemaphoreType.DMA,
    ],
)
def _sc_gather(ids_hbm, tab_hbm, out_hbm, idv, sidT, bposT,
               r0, r1, x0, x1, gs0, gs1, ws0, ws1):
    wid = lax.axis_index("s") * NC + lax.axis_index("c")
    base = wid * PER_W

    pltpu.sync_copy(ids_hbm.at[pl.ds(base, PER_W)], idv)

    zv = jnp.zeros((L,), jnp.float32)
    for k in range(EMB // L):
        r0[DUMMY, pl.ds(k * L, L)] = zv
        r1[DUMMY, pl.ds(k * L, L)] = zv

    ii = lax.iota(jnp.int32, L)

    def prep(g, c2):
        t = g * L + ii
        v = idv[pl.ds(g * L, L)]
        m = (v >= LO) & (v < HI)
        sid = jnp.where(m, v - LO, (v >> 2) & 131071)
        b = (t * 5243) >> 20
        s = t - b * 200
        dest = s * 128 + b
        bpos = jnp.where(m, b, DUMMY)
        plsc.store_scatter(sidT, [dest], sid)
        plsc.store_scatter(bposT, [dest], bpos)
        return c2

    lax.fori_loop(0, G, prep, 0)

    def gat(s, rb, sb):
        pltpu.async_copy(tab_hbm.at[sidT.at[pl.ds(s * NB, NB)]],
                         rb.at[pl.ds(0, NB)], sb)

    def wat_g(rb, sb):
        pltpu.make_async_copy(tab_hbm.at[sidT.at[pl.ds(0, NB)]],
                              rb.at[pl.ds(0, NB)], sb).wait()

    def wrt(s, xb, sb):
        pltpu.async_copy(xb, out_hbm.at[s, :, wid], sb)

    def wat_w(xb, sb):
        pltpu.make_async_copy(xb, out_hbm.at[0, :, wid], sb).wait()

    def transpose(s, rb, xb):
        bidx = [bposT[pl.ds(s * 128 + bg * L, L)] for bg in range(NB // L)]
        for bg in range(NB // L):
            for e in range(EMB):
                col = jnp.full((L,), e, jnp.int32)
                xb[e >> 3, e & 7, pl.ds(bg * L, L)] = plsc.load_gather(
                    rb, [bidx[bg], col])

    gat(0, r0, gs0)
    gat(1, r1, gs1)

    def step(g, c2):
        s0 = 2 * g
        s1 = 2 * g + 1

        wat_g(r0, gs0)

        @pl.when(s0 >= 2)
        def _():
            wat_w(x0, ws0)

        transpose(s0, r0, x0)
        wrt(s0, x0, ws0)

        @pl.when(s0 + 2 < SEQ)
        def _():
            gat(s0 + 2, r0, gs0)

        wat_g(r1, gs1)

        @pl.when(s1 >= 2)
        def _():
            wat_w(x1, ws1)

        transpose(s1, r1, x1)
        wrt(s1, x1, ws1)

        @pl.when(s1 + 2 < SEQ)
        def _():
            gat(s1 + 2, r1, gs1)

        return c2

    lax.fori_loop(0, SEQ // 2, step, 0)

    wat_w(x0, ws0)
    wat_w(x1, ws1)


def kernel(input_ids, tr):
    ids = input_ids.reshape(NTOK)
    x = _sc_gather(ids, tr)
    return x.transpose(2, 4, 0, 1, 3).reshape(BATCH, SEQ, EMB)


# speedup vs baseline: 1.4502x; 1.4502x over previous
import functools

import jax
import jax.numpy as jnp
from jax import lax
from jax.experimental import pallas as pl
from jax.experimental.pallas import tpu as pltpu
from jax.experimental.pallas import tpu_sc as plsc

VOCAB = 1_000_000
WORLD = 4
MY_RANK = 1
PART = VOCAB // WORLD
LO = MY_RANK * PART
HI = LO + PART
EMB = 64
BATCH = 4096
SEQ = 200
NTOK = BATCH * SEQ

NC = 2
NS = 16
NW = NC * NS
NB = BATCH // NW
PER_W = NB * SEQ
L = 16
G = PER_W // L
XW = 133


@functools.partial(
    pl.kernel,
    out_type=jax.ShapeDtypeStruct((SEQ, EMB // 8, BATCH // 128, 8, 128),
                                  jnp.float32),
    mesh=plsc.VectorSubcoreMesh(core_axis_name="c", subcore_axis_name="s"),
    compiler_params=pltpu.CompilerParams(
        use_tc_tiling_on_sc=False, needs_layout_passes=False),
    scratch_types=[
        pltpu.VMEM((PER_W,), jnp.int32),
        pltpu.VMEM((PER_W,), jnp.int32),
        pltpu.VMEM((PER_W,), jnp.float32),
        pltpu.VMEM((NB, EMB), jnp.float32),
        pltpu.VMEM((NB, EMB), jnp.float32),
        pltpu.VMEM((EMB, XW), jnp.float32),
        pltpu.VMEM((EMB, XW), jnp.float32),
        pltpu.SemaphoreType.DMA,
        pltpu.SemaphoreType.DMA,
        pltpu.SemaphoreType.DMA,
        pltpu.SemaphoreType.DMA,
    ],
)
def _sc_gather(ids_hbm, tab_hbm, out_hbm, idv, sidT, mskT,
               r0, r1, x0, x1, gs0, gs1, ws0, ws1):
    wid = lax.axis_index("s") * NC + lax.axis_index("c")
    base = wid * PER_W

    pltpu.sync_copy(ids_hbm.at[pl.ds(base, PER_W)], idv)

    ii = lax.iota(jnp.int32, L)
    one = jnp.ones((L,), jnp.float32)
    zero = jnp.zeros((L,), jnp.float32)

    def prep(g, c2):
        t = g * L + ii
        v = idv[pl.ds(g * L, L)]
        m = (v >= LO) & (v < HI)
        sid = jnp.where(m, v - LO, (v >> 2) & 131071)
        b = (t * 5243) >> 20
        s = t - b * 200
        dest = s * 128 + b
        plsc.store_scatter(sidT, [dest], sid)
        plsc.store_scatter(mskT, [dest], jnp.where(m, one, zero))
        return c2

    lax.fori_loop(0, G, prep, 0)

    def gat(s, rb, sb):
        pltpu.async_copy(tab_hbm.at[sidT.at[pl.ds(s * NB, NB)]], rb, sb)

    def wat_g(rb, sb):
        pltpu.make_async_copy(tab_hbm.at[sidT.at[pl.ds(0, NB)]], rb, sb).wait()

    def wrt(s, xb, sb):
        for e8 in range(EMB // 8):
            pltpu.async_copy(xb.at[pl.ds(e8 * 8, 8), pl.ds(0, 128)],
                             out_hbm.at[s, e8, wid], sb)

    def wat_w(xb, sb):
        for e8 in range(EMB // 8):
            pltpu.make_async_copy(xb.at[pl.ds(0, 8), pl.ds(0, 128)],
                                  out_hbm.at[0, e8, wid], sb).wait()

    def transpose(s, rb, xb):
        for bg in range(NB // L):
            mv = mskT[pl.ds(s * 128 + bg * L, L)]
            for j in range(L):
                b = bg * L + j
                sp = mv.at[jnp.full((L,), j, jnp.int32)].get(
                    mode="promise_in_bounds")
                colv = jnp.full((L,), b, jnp.int32)
                for k in range(EMB // L):
                    rowv = ii + (k * L)
                    plsc.store_scatter(
                        xb, [rowv, colv], rb[b, pl.ds(k * L, L)] * sp)

    gat(0, r0, gs0)
    gat(1, r1, gs1)

    def step(g, c2):
        s0 = 2 * g
        s1 = 2 * g + 1

        wat_g(r0, gs0)

        @pl.when(s0 >= 2)
        def _():
            wat_w(x0, ws0)

        transpose(s0, r0, x0)
        wrt(s0, x0, ws0)

        @pl.when(s0 + 2 < SEQ)
        def _():
            gat(s0 + 2, r0, gs0)

        wat_g(r1, gs1)

        @pl.when(s1 >= 2)
        def _():
            wat_w(x1, ws1)

        transpose(s1, r1, x1)
        wrt(s1, x1, ws1)

        @pl.when(s1 + 2 < SEQ)
        def _():
            gat(s1 + 2, r1, gs1)

        return c2

    lax.fori_loop(0, SEQ // 2, step, 0)

    wat_w(x0, ws0)
    wat_w(x1, ws1)


def kernel(input_ids, tr):
    ids = input_ids.reshape(NTOK)
    x = _sc_gather(ids, tr)
    return x.transpose(2, 4, 0, 1, 3).reshape(BATCH, SEQ, EMB)


# speedup vs baseline: 1.8580x; 1.2812x over previous
import functools

import jax
import jax.numpy as jnp
from jax import lax
from jax.experimental import pallas as pl
from jax.experimental.pallas import tpu as pltpu
from jax.experimental.pallas import tpu_sc as plsc

VOCAB = 1_000_000
WORLD = 4
MY_RANK = 1
PART = VOCAB // WORLD
LO = MY_RANK * PART
HI = LO + PART
EMB = 64
BATCH = 4096
SEQ = 200
NTOK = BATCH * SEQ

NC = 2
NS = 16
NW = NC * NS
PER_W = NTOK // NW
L = 16
C = 256
CZ = 512
NZ = PER_W // CZ
G = PER_W // L


@functools.partial(
    pl.kernel,
    out_type=jax.ShapeDtypeStruct((NTOK, EMB), jnp.float32),
    mesh=plsc.VectorSubcoreMesh(core_axis_name="c", subcore_axis_name="s"),
    compiler_params=pltpu.CompilerParams(
        use_tc_tiling_on_sc=False, needs_layout_passes=False),
    scratch_types=[
        pltpu.VMEM((PER_W,), jnp.int32),
        pltpu.VMEM((PER_W,), jnp.int32),
        pltpu.VMEM((CZ, EMB), jnp.float32),
        pltpu.VMEM((C, EMB), jnp.float32),
        pltpu.VMEM((C, EMB), jnp.float32),
        pltpu.SemaphoreType.DMA,
        pltpu.SemaphoreType.DMA,
        pltpu.SemaphoreType.DMA,
        pltpu.SemaphoreType.DMA,
        pltpu.SemaphoreType.DMA,
    ],
)
def _sc_gather(ids_hbm, tab_hbm, out_hbm, idv, dposc, zbuf, r0, r1,
               zsem, gs0, gs1, ws0, ws1):
    wid = lax.axis_index("s") * NC + lax.axis_index("c")
    base = wid * PER_W

    pltpu.sync_copy(ids_hbm.at[pl.ds(base, PER_W)], idv)

    zv = jnp.zeros((L,), jnp.float32)

    def zr(r, c2):
        for k in range(EMB // L):
            zbuf[r, pl.ds(k * L, L)] = zv
        return c2

    lax.fori_loop(0, CZ, zr, 0)

    for i in range(NZ):
        pltpu.async_copy(zbuf, out_hbm.at[pl.ds(base + i * CZ, CZ)], zsem)

    ii = lax.iota(jnp.int32, L)

    def comp(i, cnt):
        v = idv[pl.ds(i * L, L)]
        m = (v >= LO) & (v < HI)
        plsc.store_compressed(idv.at[pl.ds(cnt, L)], v - LO, mask=m)
        plsc.store_compressed(dposc.at[pl.ds(cnt, L)], (base + i * L) + ii, mask=m)
        return cnt + jnp.sum(m.astype(jnp.int32))

    cnt = lax.fori_loop(0, G, comp, jnp.int32(0))

    neg = jnp.int32(-2147483648)
    s0 = jnp.max(jnp.where(ii == 0, idv[pl.ds(0, L)], neg))
    p0 = jnp.max(jnp.where(ii == 0, dposc[pl.ds(0, L)], neg))
    sidpad = jnp.full((L,), s0, jnp.int32)
    dpospad = jnp.full((L,), p0, jnp.int32)

    @pl.when((cnt & 15) != 0)
    def _():
        gg = (cnt >> 4) << 4
        keep = (gg + ii) < cnt
        idv[pl.ds(gg, L)] = jnp.where(keep, idv[pl.ds(gg, L)], sidpad)
        dposc[pl.ds(gg, L)] = jnp.where(keep, dposc[pl.ds(gg, L)], dpospad)

    nfull = (cnt + (C - 1)) >> 8
    glo = (cnt + 15) >> 4
    ghi = nfull << 4

    def padg(g, c2):
        idv[pl.ds(g * L, L)] = sidpad
        dposc[pl.ds(g * L, L)] = dpospad
        return c2

    lax.fori_loop(glo, ghi, padg, 0)

    def gat(j, rb, sb):
        pltpu.async_copy(tab_hbm.at[idv.at[pl.ds(j * C, C)]], rb, sb)

    def wat_g(rb, sb):
        pltpu.make_async_copy(tab_hbm.at[idv.at[pl.ds(0, C)]], rb, sb).wait()

    def sca(j, rb, sb):
        pltpu.async_copy(rb, out_hbm.at[dposc.at[pl.ds(j * C, C)]], sb)

    def wat_w(rb, sb):
        pltpu.make_async_copy(rb, out_hbm.at[dposc.at[pl.ds(0, C)]], sb).wait()

    @pl.when(nfull > 0)
    def _():
        gat(0, r0, gs0)

    @pl.when(nfull > 1)
    def _():
        gat(1, r1, gs1)

    for i in range(NZ):
        pltpu.make_async_copy(zbuf, out_hbm.at[pl.ds(base, CZ)], zsem).wait()

    def step(g, c2):
        j0 = 2 * g
        j1 = 2 * g + 1

        @pl.when(j0 < nfull)
        def _():
            wat_g(r0, gs0)
            sca(j0, r0, ws0)

        @pl.when(j1 < nfull)
        def _():
            wat_g(r1, gs1)
            sca(j1, r1, ws1)

        @pl.when(j0 + 2 < nfull)
        def _():
            wat_w(r0, ws0)
            gat(j0 + 2, r0, gs0)

        @pl.when(j1 + 2 < nfull)
        def _():
            wat_w(r1, ws1)
            gat(j1 + 2, r1, gs1)

        return c2

    lax.fori_loop(0, (nfull + 1) >> 1, step, 0)

    @pl.when(nfull > 0)
    def _():
        wat_w(r0, ws0)

    @pl.when(nfull > 1)
    def _():
        wat_w(r1, ws1)


def kernel(input_ids, tr):
    ids = input_ids.reshape(NTOK)
    out = _sc_gather(ids, tr)
    return out.reshape(BATCH, SEQ, EMB)
